# baseline (speedup 1.0000x reference)
import jax
import jax.numpy as jnp
from jax import lax
from jax.experimental import pallas as pl
from jax.experimental.pallas import tpu as pltpu
from jax.experimental.pallas import tpu_sc as plsc

B, S = 1024, 20
N = B * S
N2V_DIM = 128
T_FEAT, ED = 4, 8
T2V_DIM = 32
SMALL_D = 16
SMALL_V = 1000
TARGET = 64

NC, NS = 2, 16
NW = NC * NS
TOK_PER_W = N // NW
CHUNK = 128
NCHUNK = TOK_PER_W // CHUNK
NGRP = TOK_PER_W // 16


def _sc_gather_body(n2v_hbm, bt_hbm, ev_hbm, eq_hbm,
                    idx_n2v_hbm, idx_bt_hbm, idx_ev_hbm, idx_eq_hbm,
                    out_spat, out_bt, out_ev, out_eq,
                    idx_v, idxs_v, big_v, tab_v, outt_v, sem):
    wid = lax.axis_index("s") * NC + lax.axis_index("c")
    base = wid * TOK_PER_W

    pltpu.sync_copy(idx_n2v_hbm.at[pl.ds(base, TOK_PER_W)], idx_v)
    big_copies = [
        pltpu.async_copy(
            n2v_hbm.at[idx_v.at[pl.ds(j * CHUNK, CHUNK)]],
            big_v.at[pl.ds(j * CHUNK, CHUNK)], sem)
        for j in range(NCHUNK)
    ]

    def small_table(tab_hbm, idx_hbm, out_hbm):
        pltpu.sync_copy(tab_hbm, tab_v)
        pltpu.sync_copy(idx_hbm.at[pl.ds(base, TOK_PER_W)], idxs_v)

        def grp(g, _):
            off = pl.multiple_of(g * 16, 16)
            ids = idxs_v[pl.ds(off, 16)] * SMALL_D
            for k in range(SMALL_D):
                outt_v[k, pl.ds(off, 16)] = plsc.load_gather(tab_v, [ids + k])
            return _

        lax.fori_loop(0, NGRP, grp, None)
        pltpu.sync_copy(outt_v, out_hbm.at[:, pl.ds(base, TOK_PER_W)])

    small_table(bt_hbm, idx_bt_hbm, out_bt)
    small_table(ev_hbm, idx_ev_hbm, out_ev)
    small_table(eq_hbm, idx_eq_hbm, out_eq)

    for j, c in enumerate(big_copies):
        c.wait()
        pltpu.sync_copy(big_v.at[pl.ds(j * CHUNK, CHUNK)],
                        out_spat.at[pl.ds(base + j * CHUNK, CHUNK)])


def _sc_gather(n2v, btab, etab, qtab, idx_n2v, idx_bt, idx_ev, idx_eq):
    mesh = plsc.VectorSubcoreMesh(core_axis_name="c", subcore_axis_name="s")
    fn = pl.kernel(
        _sc_gather_body,
        out_type=[
            jax.ShapeDtypeStruct((N, N2V_DIM), jnp.float32),
            jax.ShapeDtypeStruct((SMALL_D, N), jnp.float32),
            jax.ShapeDtypeStruct((SMALL_D, N), jnp.float32),
            jax.ShapeDtypeStruct((SMALL_D, N), jnp.float32),
        ],
        mesh=mesh,
        compiler_params=pltpu.CompilerParams(needs_layout_passes=False),
        scratch_types=[
            pltpu.VMEM((TOK_PER_W,), jnp.int32),
            pltpu.VMEM((TOK_PER_W,), jnp.int32),
            pltpu.VMEM((TOK_PER_W, N2V_DIM), jnp.float32),
            pltpu.VMEM((SMALL_V * SMALL_D,), jnp.float32),
            pltpu.VMEM((SMALL_D, TOK_PER_W), jnp.float32),
            pltpu.SemaphoreType.DMA,
        ],
    )
    return fn(n2v, btab, etab, qtab, idx_n2v, idx_bt, idx_ev, idx_eq)


_PI_HI = 3.14159274101257324
_PI_LO = -8.74227765734758577e-08


def _fast_sin(x):
    n = jnp.round(x * (1.0 / 3.14159265358979))
    r = x - n * _PI_HI
    r = r - n * _PI_LO
    r = jnp.clip(r, -1.6, 1.6)
    s = r * r
    p = -2.50507586e-08
    p = p * s + 2.75573143e-06
    p = p * s + -1.98412701e-04
    p = p * s + 8.33333377e-03
    p = p * s + -1.66666672e-01
    p = r + r * (s * p)
    odd = jnp.round(n * 0.5) * 2.0 != n
    return jnp.where(odd, -p, p)


SB = 4


def _tc_body(spat_ref, btt_ref, evt_ref, eqt_ref, tf_ref, popc_ref,
             counts_ref, wf_ref, bf_ref, popw_ref, popb_ref,
             w1t_ref, w2t_ref, w3t_ref, w4t_ref, w5t_ref, w6t_ref,
             projb_ref, out_ref):
    f32 = jnp.float32

    pw2 = lax.dot_general(w4t_ref[...], popw_ref[...],
                          (((1,), (0,)), ((), ())),
                          preferred_element_type=f32)
    pb2 = lax.dot_general(w4t_ref[...], popb_ref[...],
                          (((1,), (0,)), ((), ())),
                          preferred_element_type=f32)

    cR = (((1,), (0,)), ((), ()))
    cT = (((1,), (1,)), ((), ()))

    for u in range(SB):
        c = pl.ds(u * B, B)

        tf_blk = tf_ref[:, c]
        tfr = jnp.concatenate(
            [jnp.broadcast_to(tf_blk[t:t + 1, :], (ED, B))
             for t in range(T_FEAT)], axis=0)
        aff = tfr * wf_ref[...] + bf_ref[...]
        row = lax.broadcasted_iota(jnp.int32, (T2V_DIM, B), 0)
        temporal = jnp.where(row % ED == 0, aff, _fast_sin(aff))

        csum_s = jnp.sum(counts_ref[u])

        acc = lax.dot_general(w1t_ref[...], spat_ref[pl.ds(u * B, B), :],
                              cT, preferred_element_type=f32)
        acc += lax.dot_general(w2t_ref[...], temporal, cR,
                               preferred_element_type=f32)
        acc += lax.dot_general(w3t_ref[...], btt_ref[:, c], cR,
                               preferred_element_type=f32) * csum_s
        acc += pw2 * popc_ref[:, c] + pb2
        acc += lax.dot_general(w5t_ref[...], evt_ref[:, c], cR,
                               preferred_element_type=f32)
        acc += lax.dot_general(w6t_ref[...], eqt_ref[:, c], cR,
                               preferred_element_type=f32)
        out_ref[u] = acc + projb_ref[...]


def _tc_fused(spat, btt, evt, eqt, tfT, popcT, countsT,
              wf, bf, popw, popb, w1t, w2t, w3t, w4t, w5t, w6t, projb):
    grid = (S // SB,)
    col_spec = lambda h: pl.BlockSpec((h, SB * B), lambda i: (0, i))
    full = lambda a: pl.BlockSpec(a.shape, lambda i: (0,) * a.ndim)
    return pl.pallas_call(
        _tc_body,
        grid=grid,
        in_specs=[
            pl.BlockSpec((SB * B, N2V_DIM), lambda i: (i, 0)),
            col_spec(SMALL_D), col_spec(SMALL_D), col_spec(SMALL_D),
            col_spec(T_FEAT), col_spec(1),
            pl.BlockSpec((SB, 1, B), lambda i: (i, 0, 0)),
            full(wf), full(bf), full(popw), full(popb),
            full(w1t), full(w2t), full(w3t), full(w4t), full(w5t),
            full(w6t), full(projb),
        ],
        out_specs=pl.BlockSpec((SB, TARGET, B), lambda i: (i, 0, 0)),
        out_shape=jax.ShapeDtypeStruct((S, TARGET, B), jnp.float32),
        compiler_params=pltpu.CompilerParams(
            fuse_transposed_lhs_in_matmul=True),
    )(spat, btt, evt, eqt, tfT, popcT, countsT,
      wf, bf, popw, popb, w1t, w2t, w3t, w4t, w5t, w6t, projb)


def _prep_body(a_ref, b_ref, c_ref, d_ref, oa, ob, oc, od):
    oa[...] = a_ref[...].T
    ob[...] = b_ref[..., 0].T
    oc[...] = c_ref[...].T
    od[...] = d_ref[...].T


def _prep_idx(a, b3, c, d):
    full = lambda x: pl.BlockSpec(x.shape, lambda: (0,) * x.ndim)
    ot = jax.ShapeDtypeStruct((S, B), jnp.int32)
    return pl.pallas_call(
        _prep_body,
        in_specs=[full(a), full(b3), full(c), full(d)],
        out_specs=[full(ot), full(ot), full(ot), full(ot)],
        out_shape=[ot, ot, ot, ot],
    )(a, b3, c, d)


def kernel(neighborhood_ids, time_features, building_type_ids,
           building_counts, population, event_type_ids, equipment_ids,
           node2vec_table, t2v_weight, t2v_bias, building_table,
           pop_W, pop_b, event_table, equip_table, proj_W, proj_b):
    i0, i1, i2, i3 = _prep_idx(
        neighborhood_ids.reshape(B, S), building_type_ids,
        event_type_ids.reshape(B, S), equipment_ids.reshape(B, S))
    idx_n2v = i0.reshape(N)
    idx_bt = i1.reshape(N)
    idx_ev = i2.reshape(N)
    idx_eq = i3.reshape(N)

    spat, btt, evt, eqt = _sc_gather(
        node2vec_table,
        building_table.reshape(SMALL_V * SMALL_D),
        event_table.reshape(SMALL_V * SMALL_D),
        equip_table.reshape(SMALL_V * SMALL_D),
        idx_n2v, idx_bt, idx_ev, idx_eq)

    wt = proj_W.T
    out = _tc_fused(
        spat, btt, evt, eqt,
        jnp.transpose(time_features, (2, 1, 0)).reshape(T_FEAT, N),
        population[:, :, 0].T.reshape(1, N),
        building_counts[:, :, 0].T.reshape(S, 1, B),
        t2v_weight.reshape(T2V_DIM, 1),
        t2v_bias.reshape(T2V_DIM, 1),
        pop_W.reshape(8, 1), pop_b.reshape(8, 1),
        wt[:, 0:128], wt[:, 128:160], wt[:, 160:176],
        wt[:, 176:184], wt[:, 184:200], wt[:, 200:216],
        proj_b.reshape(TARGET, 1))
    return jnp.transpose(out, (2, 0, 1))


# speedup vs baseline: 1.2855x; 1.2855x over previous
import jax
import jax.numpy as jnp
from jax import lax
from jax.experimental import pallas as pl
from jax.experimental.pallas import tpu as pltpu
from jax.experimental.pallas import tpu_sc as plsc

B, S = 1024, 20
N = B * S
N2V_DIM = 128
T_FEAT, ED = 4, 8
T2V_DIM = 32
SMALL_D = 16
SMALL_V = 1000
TARGET = 64

NC, NS = 2, 16
NW = NC * NS
TOK_PER_W = N // NW
CHUNK = 128
NCHUNK = TOK_PER_W // CHUNK
NGRP = TOK_PER_W // 16


def _sc_gather_body(n2v_hbm, bt_hbm, ev_hbm, eq_hbm,
                    idx_n2v_hbm, idx_bt_hbm, idx_ev_hbm, idx_eq_hbm,
                    out_spat, out_bt, out_ev, out_eq,
                    idx_v, idxs_v, big_v, tab_v, outt_v, sem):
    wid = lax.axis_index("s") * NC + lax.axis_index("c")
    base = wid * TOK_PER_W

    pltpu.sync_copy(idx_n2v_hbm.at[pl.ds(base, TOK_PER_W)], idx_v)
    big_copies = [
        pltpu.async_copy(
            n2v_hbm.at[idx_v.at[pl.ds(j * CHUNK, CHUNK)]],
            big_v.at[pl.ds(j * CHUNK, CHUNK)], sem)
        for j in range(NCHUNK)
    ]

    def small_table(tab_hbm, idx_hbm, out_hbm):
        pltpu.sync_copy(tab_hbm, tab_v)
        pltpu.sync_copy(idx_hbm.at[pl.ds(base, TOK_PER_W)], idxs_v)

        def grp(g, _):
            off = pl.multiple_of(g * 16, 16)
            ids = idxs_v[pl.ds(off, 16)] * SMALL_D
            for k in range(SMALL_D):
                outt_v[k, pl.ds(off, 16)] = plsc.load_gather(tab_v, [ids + k])
            return _

        lax.fori_loop(0, NGRP, grp, None)
        pltpu.sync_copy(outt_v, out_hbm.at[:, pl.ds(base, TOK_PER_W)])

    small_table(bt_hbm, idx_bt_hbm, out_bt)
    small_table(ev_hbm, idx_ev_hbm, out_ev)
    small_table(eq_hbm, idx_eq_hbm, out_eq)

    for j, c in enumerate(big_copies):
        c.wait()
        pltpu.sync_copy(big_v.at[pl.ds(j * CHUNK, CHUNK)],
                        out_spat.at[pl.ds(base + j * CHUNK, CHUNK)])


def _sc_gather(n2v, btab, etab, qtab, idx_n2v, idx_bt, idx_ev, idx_eq):
    mesh = plsc.VectorSubcoreMesh(core_axis_name="c", subcore_axis_name="s")
    fn = pl.kernel(
        _sc_gather_body,
        out_type=[
            jax.ShapeDtypeStruct((N, N2V_DIM), jnp.float32),
            jax.ShapeDtypeStruct((SMALL_D, N), jnp.float32),
            jax.ShapeDtypeStruct((SMALL_D, N), jnp.float32),
            jax.ShapeDtypeStruct((SMALL_D, N), jnp.float32),
        ],
        mesh=mesh,
        compiler_params=pltpu.CompilerParams(needs_layout_passes=False),
        scratch_types=[
            pltpu.VMEM((TOK_PER_W,), jnp.int32),
            pltpu.VMEM((TOK_PER_W,), jnp.int32),
            pltpu.VMEM((TOK_PER_W, N2V_DIM), jnp.float32),
            pltpu.VMEM((SMALL_V * SMALL_D,), jnp.float32),
            pltpu.VMEM((SMALL_D, TOK_PER_W), jnp.float32),
            pltpu.SemaphoreType.DMA,
        ],
    )
    return fn(n2v, btab, etab, qtab, idx_n2v, idx_bt, idx_ev, idx_eq)


_PI_HI = 3.14159274101257324
_PI_LO = -8.74227765734758577e-08


def _fast_sin(x):
    n = jnp.round(x * (1.0 / 3.14159265358979))
    r = x - n * _PI_HI
    r = r - n * _PI_LO
    r = jnp.clip(r, -1.6, 1.6)
    s = r * r
    p = -2.50507586e-08
    p = p * s + 2.75573143e-06
    p = p * s + -1.98412701e-04
    p = p * s + 8.33333377e-03
    p = p * s + -1.66666672e-01
    p = r + r * (s * p)
    odd = jnp.round(n * 0.5) * 2.0 != n
    return jnp.where(odd, -p, p)


SB = 4


def _tc_body(spat_ref, btt_ref, evt_ref, eqt_ref, tf_ref, popc_ref,
             counts_ref, wf_ref, bf_ref, popw_ref, popb_ref,
             w1t_ref, w2t_ref, w3t_ref, w4t_ref, w5t_ref, w6t_ref,
             projb_ref, out_ref):
    f32 = jnp.float32

    pw2 = lax.dot_general(w4t_ref[...], popw_ref[...],
                          (((1,), (0,)), ((), ())),
                          preferred_element_type=f32)
    pb2 = lax.dot_general(w4t_ref[...], popb_ref[...],
                          (((1,), (0,)), ((), ())),
                          preferred_element_type=f32)

    cR = (((1,), (0,)), ((), ()))
    cT = (((1,), (1,)), ((), ()))

    for u in range(SB):
        c = pl.ds(u * B, B)

        tf_blk = tf_ref[:, c]
        tfr = jnp.concatenate(
            [jnp.broadcast_to(tf_blk[t:t + 1, :], (ED, B))
             for t in range(T_FEAT)], axis=0)
        aff = tfr * wf_ref[...] + bf_ref[...]
        row = lax.broadcasted_iota(jnp.int32, (T2V_DIM, B), 0)
        temporal = jnp.where(row % ED == 0, aff, _fast_sin(aff))

        csum_s = jnp.sum(counts_ref[u])

        acc = lax.dot_general(w1t_ref[...], spat_ref[pl.ds(u * B, B), :],
                              cT, preferred_element_type=f32)
        acc += lax.dot_general(w2t_ref[...], temporal, cR,
                               preferred_element_type=f32)
        acc += lax.dot_general(w3t_ref[...], btt_ref[:, c], cR,
                               preferred_element_type=f32) * csum_s
        acc += pw2 * popc_ref[:, c] + pb2
        acc += lax.dot_general(w5t_ref[...], evt_ref[:, c], cR,
                               preferred_element_type=f32)
        acc += lax.dot_general(w6t_ref[...], eqt_ref[:, c], cR,
                               preferred_element_type=f32)
        out_ref[u] = acc + projb_ref[...]


def _tc_fused(spat, btt, evt, eqt, tfT, popcT, countsT,
              wf, bf, popw, popb, w1t, w2t, w3t, w4t, w5t, w6t, projb):
    grid = (S // SB,)
    col_spec = lambda h: pl.BlockSpec((h, SB * B), lambda i: (0, i))
    full = lambda a: pl.BlockSpec(a.shape, lambda i: (0,) * a.ndim)
    return pl.pallas_call(
        _tc_body,
        grid=grid,
        in_specs=[
            pl.BlockSpec((SB * B, N2V_DIM), lambda i: (i, 0)),
            col_spec(SMALL_D), col_spec(SMALL_D), col_spec(SMALL_D),
            col_spec(T_FEAT), col_spec(1),
            pl.BlockSpec((SB, 1, B), lambda i: (i, 0, 0)),
            full(wf), full(bf), full(popw), full(popb),
            full(w1t), full(w2t), full(w3t), full(w4t), full(w5t),
            full(w6t), full(projb),
        ],
        out_specs=pl.BlockSpec((SB, TARGET, B), lambda i: (i, 0, 0)),
        out_shape=jax.ShapeDtypeStruct((S, TARGET, B), jnp.float32),
        compiler_params=pltpu.CompilerParams(
            fuse_transposed_lhs_in_matmul=True),
    )(spat, btt, evt, eqt, tfT, popcT, countsT,
      wf, bf, popw, popb, w1t, w2t, w3t, w4t, w5t, w6t, projb)


def kernel(neighborhood_ids, time_features, building_type_ids,
           building_counts, population, event_type_ids, equipment_ids,
           node2vec_table, t2v_weight, t2v_bias, building_table,
           pop_W, pop_b, event_table, equip_table, proj_W, proj_b):
    idx_n2v = neighborhood_ids.reshape(B, S).T.reshape(N)
    idx_bt = building_type_ids[:, :, 0].T.reshape(N)
    idx_ev = event_type_ids.reshape(B, S).T.reshape(N)
    idx_eq = equipment_ids.reshape(B, S).T.reshape(N)

    spat, btt, evt, eqt = _sc_gather(
        node2vec_table,
        building_table.reshape(SMALL_V * SMALL_D),
        event_table.reshape(SMALL_V * SMALL_D),
        equip_table.reshape(SMALL_V * SMALL_D),
        idx_n2v, idx_bt, idx_ev, idx_eq)

    wt = proj_W.T
    out = _tc_fused(
        spat, btt, evt, eqt,
        jnp.transpose(time_features, (2, 1, 0)).reshape(T_FEAT, N),
        population[:, :, 0].T.reshape(1, N),
        building_counts[:, :, 0].T.reshape(S, 1, B),
        t2v_weight.reshape(T2V_DIM, 1),
        t2v_bias.reshape(T2V_DIM, 1),
        pop_W.reshape(8, 1), pop_b.reshape(8, 1),
        wt[:, 0:128], wt[:, 128:160], wt[:, 160:176],
        wt[:, 176:184], wt[:, 184:200], wt[:, 200:216],
        proj_b.reshape(TARGET, 1))
    return jnp.transpose(out, (2, 0, 1))
